# baseline (speedup 1.0000x reference)
import functools

import jax
import jax.numpy as jnp
from jax import lax
from jax.experimental import pallas as pl
from jax.experimental.pallas import tpu as pltpu
from jax.experimental.pallas import tpu_sc as plsc

NC = 2
NS = 16
L = 16
NW = NC * NS
K = 128


def _sc_mesh():
    return plsc.VectorSubcoreMesh(
        core_axis_name="c", subcore_axis_name="s", num_cores=NC,
        num_subcores=NS)


def _make_deg_kernel(cpt, n_pad):
    @functools.partial(
        pl.kernel,
        out_type=jax.ShapeDtypeStruct((2 * NW, n_pad), jnp.float32),
        mesh=_sc_mesh(),
        compiler_params=pltpu.CompilerParams(needs_layout_passes=False),
        scratch_types=[
            pltpu.VMEM((cpt, K), jnp.int32),
            pltpu.VMEM((cpt, K), jnp.int32),
            pltpu.VMEM((n_pad,), jnp.float32),
            pltpu.VMEM((n_pad,), jnp.float32),
        ],
    )
    def deg_kernel(src_hbm, dst_hbm, out_hbm, src_v, dst_v, hs_v, hd_v):
        cid = lax.axis_index("c")
        sid = lax.axis_index("s")
        wid = sid * NC + cid
        pltpu.sync_copy(src_hbm.at[wid], src_v)
        pltpu.sync_copy(dst_hbm.at[wid], dst_v)
        zero = jnp.zeros((L,), jnp.float32)

        @pl.loop(0, n_pad // L)
        def _(i):
            hs_v[pl.ds(i * L, L)] = zero
            hd_v[pl.ds(i * L, L)] = zero

        ones = jnp.ones((L,), jnp.float32)

        @pl.loop(0, cpt)
        def _(j):
            @pl.loop(0, K // L)
            def _(k):
                si = src_v[j, pl.ds(k * L, L)]
                di = dst_v[j, pl.ds(k * L, L)]
                plsc.addupdate_scatter(hs_v, [si], ones)
                plsc.addupdate_scatter(hd_v, [di], ones)

        pltpu.sync_copy(hs_v, out_hbm.at[2 * wid])
        pltpu.sync_copy(hd_v, out_hbm.at[2 * wid + 1])

    return deg_kernel


def _make_mp_kernel(cpt, n_pad, f):
    rows_per_tile = n_pad // NS
    nzc = rows_per_tile // K

    @functools.partial(
        pl.kernel,
        out_type=jax.ShapeDtypeStruct((NC * n_pad, f), jnp.float32),
        mesh=_sc_mesh(),
        compiler_params=pltpu.CompilerParams(needs_layout_passes=False),
        scratch_types=[
            pltpu.VMEM((cpt, K), jnp.int32),
            pltpu.VMEM((cpt, K), jnp.int32),
            pltpu.VMEM((K, f), jnp.float32),
            pltpu.VMEM_SHARED((n_pad, f), jnp.float32),
            pltpu.SemaphoreType.DMA,
        ],
    )
    def mp_kernel(hs_hbm, src_hbm, dst_hbm, out_hbm, src_v, dst_v, buf,
                  agg_sh, sem):
        cid = lax.axis_index("c")
        sid = lax.axis_index("s")
        wid = sid * NC + cid
        nb = cpt
        pltpu.sync_copy(src_hbm.at[wid], src_v)
        pltpu.sync_copy(dst_hbm.at[wid], dst_v)

        zero = jnp.zeros((L,), jnp.float32)

        @pl.loop(0, K)
        def _(r):
            @pl.loop(0, f // L)
            def _(c2):
                buf[r, pl.ds(c2 * L, L)] = zero

        @pl.loop(0, nzc)
        def _(i):
            pltpu.sync_copy(
                buf, agg_sh.at[pl.ds(sid * rows_per_tile + i * K, K)])

        plsc.subcore_barrier()

        @pl.loop(0, nb)
        def _(j):
            pltpu.async_copy(hs_hbm.at[src_v.at[j]], buf, sem).wait()
            pltpu.sync_copy(buf, agg_sh.at[dst_v.at[j]], add=True)

        plsc.subcore_barrier()

        @pl.loop(0, nzc)
        def _(i):
            base = sid * rows_per_tile + i * K
            pltpu.sync_copy(agg_sh.at[pl.ds(base, K)],
                            out_hbm.at[pl.ds(cid * n_pad + base, K)])

    return mp_kernel


def _norm_kernel(deg_part):
    def body(d_ref, o_ref):
        deg = jnp.sum(d_ref[...], axis=0)
        o_ref[...] = jnp.where(
            deg > 0.0, lax.rsqrt(jnp.maximum(deg, 1.0)), 0.0)

    return pl.pallas_call(
        body,
        out_shape=jax.ShapeDtypeStruct(deg_part.shape[1:], jnp.float32),
    )(deg_part)


def _mm_scale(x, w, scale, n_pad, rb):
    n_nodes, f = x.shape

    def body(x_ref, w_ref, s_ref, o_ref):
        i = pl.program_id(0)
        r = jnp.dot(x_ref[...], w_ref[...],
                    precision=lax.Precision.HIGHEST,
                    preferred_element_type=jnp.float32) * s_ref[...]
        rid = lax.broadcasted_iota(jnp.int32, (rb, 1), 0) + i * rb
        o_ref[...] = jnp.where(rid < n_nodes, r, 0.0)

    return pl.pallas_call(
        body,
        grid=(n_pad // rb,),
        in_specs=[
            pl.BlockSpec((rb, f), lambda i: (i, 0)),
            pl.BlockSpec((f, f), lambda i: (0, 0)),
            pl.BlockSpec((rb, 1), lambda i: (i, 0)),
        ],
        out_specs=pl.BlockSpec((rb, f), lambda i: (i, 0)),
        out_shape=jax.ShapeDtypeStruct((n_pad, f), jnp.float32),
    )(x, w, scale)


def _layer2_input(agg, norm_dst, b, w, norm_src, rb):
    _, n_pad, f = agg.shape

    def body(a_ref, nd_ref, b_ref, w_ref, ns_ref, o_ref):
        h = jnp.maximum(
            (a_ref[0] + a_ref[1]) * nd_ref[...] + b_ref[...], 0.0)
        o_ref[...] = jnp.dot(
            h, w_ref[...], precision=lax.Precision.HIGHEST,
            preferred_element_type=jnp.float32) * ns_ref[...]

    return pl.pallas_call(
        body,
        grid=(n_pad // rb,),
        in_specs=[
            pl.BlockSpec((NC, rb, f), lambda i: (0, i, 0)),
            pl.BlockSpec((rb, 1), lambda i: (i, 0)),
            pl.BlockSpec((1, f), lambda i: (0, 0)),
            pl.BlockSpec((f, f), lambda i: (0, 0)),
            pl.BlockSpec((rb, 1), lambda i: (i, 0)),
        ],
        out_specs=pl.BlockSpec((rb, f), lambda i: (i, 0)),
        out_shape=jax.ShapeDtypeStruct((n_pad, f), jnp.float32),
    )(agg, norm_dst, b, w, norm_src)


def _head(agg, norm_dst, b, wd, bd, n_nodes, rb):
    _, n_pad, f = agg.shape
    grid = n_pad // rb

    def body(a_ref, nd_ref, b_ref, wd_ref, bd_ref, o_ref, acc_ref):
        i = pl.program_id(0)
        h = jnp.maximum(
            (a_ref[0] + a_ref[1]) * nd_ref[...] + b_ref[...], 0.0)
        rid = lax.broadcasted_iota(jnp.int32, (rb, f), 0) + i * rb
        h = jnp.where(rid < n_nodes, h, 0.0)
        p = jnp.sum(h, axis=0, keepdims=True)

        @pl.when(i == 0)
        def _():
            acc_ref[...] = p

        @pl.when(i > 0)
        def _():
            acc_ref[...] = acc_ref[...] + p

        @pl.when(i == grid - 1)
        def _():
            g = acc_ref[...] * (1.0 / n_nodes)
            o_ref[...] = jnp.dot(
                g, wd_ref[...], precision=lax.Precision.HIGHEST,
                preferred_element_type=jnp.float32
            ) + bd_ref[...]

    return pl.pallas_call(
        body,
        grid=(grid,),
        in_specs=[
            pl.BlockSpec((NC, rb, f), lambda i: (0, i, 0)),
            pl.BlockSpec((rb, 1), lambda i: (i, 0)),
            pl.BlockSpec((1, f), lambda i: (0, 0)),
            pl.BlockSpec((f, 1), lambda i: (0, 0)),
            pl.BlockSpec((1, 1), lambda i: (0, 0)),
        ],
        out_specs=pl.BlockSpec((1, 1), lambda i: (0, 0)),
        out_shape=jax.ShapeDtypeStruct((1, 1), jnp.float32),
        scratch_shapes=[pltpu.VMEM((1, f), jnp.float32)],
    )(agg, norm_dst, b, wd, bd)


def kernel(in_feat, edge_index, W1, b1, W2, b2, Wd, bd):
    n_nodes, f = in_feat.shape
    n_edges = edge_index.shape[1]

    n_pad = ((n_nodes + 1 + NS * K - 1) // (NS * K)) * (NS * K)
    rb = 1024 if n_pad % 1024 == 0 else NS * K

    def _pad_dst(n):
        return n_nodes + (jnp.arange(n, dtype=jnp.int32)
                          % (n_pad - n_nodes))

    cpt = (n_edges + NW * K - 1) // (NW * K)
    e_pad = NW * K * cpt
    pad_idx = jnp.full((e_pad - n_edges,), n_nodes, jnp.int32)
    src = jnp.concatenate([edge_index[0], pad_idx]).reshape(NW, cpt, K)
    dst = jnp.concatenate(
        [edge_index[1], _pad_dst(e_pad - n_edges)]).reshape(NW, cpt, K)


    deg_part = _make_deg_kernel(cpt, n_pad)(src, dst)
    norms = _norm_kernel(deg_part.reshape(NW, 2, n_pad))
    norm_src = norms[0].reshape(n_pad, 1)
    norm_dst = norms[1].reshape(n_pad, 1)

    mp = _make_mp_kernel(cpt, n_pad, f)

    hs1 = _mm_scale(in_feat, W1, norm_src, n_pad, rb)
    agg1 = mp(hs1, src, dst).reshape(NC, n_pad, f)
    hs2 = _layer2_input(agg1, norm_dst, b1.reshape(1, f), W2, norm_src, rb)
    agg2 = mp(hs2, src, dst).reshape(NC, n_pad, f)
    out = _head(agg2, norm_dst, b2.reshape(1, f), Wd, bd.reshape(1, 1),
                n_nodes, rb)
    return out.reshape(())


# speedup vs baseline: 1.0531x; 1.0531x over previous
import functools

import jax
import jax.numpy as jnp
from jax import lax
from jax.experimental import pallas as pl
from jax.experimental.pallas import tpu as pltpu
from jax.experimental.pallas import tpu_sc as plsc

NC = 2
NS = 16
L = 16
NW = NC * NS
K = 128


def _sc_mesh():
    return plsc.VectorSubcoreMesh(
        core_axis_name="c", subcore_axis_name="s", num_cores=NC,
        num_subcores=NS)


def _make_deg_kernel(cpt, n_pad):
    @functools.partial(
        pl.kernel,
        out_type=jax.ShapeDtypeStruct((2 * NW, n_pad), jnp.float32),
        mesh=_sc_mesh(),
        compiler_params=pltpu.CompilerParams(needs_layout_passes=False),
        scratch_types=[
            pltpu.VMEM((cpt, K), jnp.int32),
            pltpu.VMEM((cpt, K), jnp.int32),
            pltpu.VMEM((n_pad,), jnp.float32),
            pltpu.VMEM((n_pad,), jnp.float32),
        ],
    )
    def deg_kernel(src_hbm, dst_hbm, out_hbm, src_v, dst_v, hs_v, hd_v):
        cid = lax.axis_index("c")
        sid = lax.axis_index("s")
        wid = sid * NC + cid
        pltpu.sync_copy(src_hbm.at[wid], src_v)
        pltpu.sync_copy(dst_hbm.at[wid], dst_v)
        zero = jnp.zeros((L,), jnp.float32)

        @pl.loop(0, n_pad // L)
        def _(i):
            hs_v[pl.ds(i * L, L)] = zero
            hd_v[pl.ds(i * L, L)] = zero

        ones = jnp.ones((L,), jnp.float32)

        @pl.loop(0, cpt)
        def _(j):
            @pl.loop(0, K // L)
            def _(k):
                si = src_v[j, pl.ds(k * L, L)]
                di = dst_v[j, pl.ds(k * L, L)]
                plsc.addupdate_scatter(hs_v, [si], ones)
                plsc.addupdate_scatter(hd_v, [di], ones)

        pltpu.sync_copy(hs_v, out_hbm.at[2 * wid])
        pltpu.sync_copy(hd_v, out_hbm.at[2 * wid + 1])

    return deg_kernel


def _make_mp_kernel(cpt, n_pad, f):
    rows_per_tile = n_pad // NS
    nzc = rows_per_tile // K

    @functools.partial(
        pl.kernel,
        out_type=jax.ShapeDtypeStruct((NC * n_pad, f), jnp.float32),
        mesh=_sc_mesh(),
        compiler_params=pltpu.CompilerParams(needs_layout_passes=False),
        scratch_types=[
            pltpu.VMEM((cpt, K), jnp.int32),
            pltpu.VMEM((cpt, K), jnp.int32),
            pltpu.VMEM((K, f), jnp.float32),
            pltpu.VMEM_SHARED((n_pad, f), jnp.float32),
            pltpu.SemaphoreType.DMA,
        ],
    )
    def mp_kernel(hs_hbm, src_hbm, dst_hbm, out_hbm, src_v, dst_v, buf,
                  agg_sh, sem):
        cid = lax.axis_index("c")
        sid = lax.axis_index("s")
        wid = sid * NC + cid
        nb = cpt
        pltpu.sync_copy(src_hbm.at[wid], src_v)
        pltpu.sync_copy(dst_hbm.at[wid], dst_v)

        zero = jnp.zeros((L,), jnp.float32)

        @pl.loop(0, K)
        def _(r):
            @pl.loop(0, f // L)
            def _(c2):
                buf[r, pl.ds(c2 * L, L)] = zero

        @pl.loop(0, nzc)
        def _(i):
            pltpu.sync_copy(
                buf, agg_sh.at[pl.ds(sid * rows_per_tile + i * K, K)])

        plsc.subcore_barrier()

        @pl.loop(0, nb)
        def _(j):
            pltpu.async_copy(hs_hbm.at[src_v.at[j]], buf, sem).wait()
            pltpu.sync_copy(buf, agg_sh.at[dst_v.at[j]], add=True)

        plsc.subcore_barrier()

        @pl.loop(0, nzc)
        def _(i):
            base = sid * rows_per_tile + i * K
            pltpu.sync_copy(agg_sh.at[pl.ds(base, K)],
                            out_hbm.at[pl.ds(cid * n_pad + base, K)])

    return mp_kernel


def _norm_kernel(deg_part):
    def body(d_ref, o_ref):
        deg = jnp.sum(d_ref[...], axis=0)
        o_ref[...] = jnp.where(
            deg > 0.0, lax.rsqrt(jnp.maximum(deg, 1.0)), 0.0)

    return pl.pallas_call(
        body,
        out_shape=jax.ShapeDtypeStruct(deg_part.shape[1:], jnp.float32),
    )(deg_part)


def _mm_scale(x, w, scale, n_pad, rb):
    f = x.shape[1]

    def body(x_ref, w_ref, s_ref, o_ref):
        o_ref[...] = jnp.dot(x_ref[...], w_ref[...],
                             precision=lax.Precision.HIGHEST,
                             preferred_element_type=jnp.float32) * s_ref[...]

    return pl.pallas_call(
        body,
        grid=(n_pad // rb,),
        in_specs=[
            pl.BlockSpec((rb, f), lambda i: (i, 0)),
            pl.BlockSpec((f, f), lambda i: (0, 0)),
            pl.BlockSpec((rb, 1), lambda i: (i, 0)),
        ],
        out_specs=pl.BlockSpec((rb, f), lambda i: (i, 0)),
        out_shape=jax.ShapeDtypeStruct((n_pad, f), jnp.float32),
    )(x, w, scale)


def _layer2_input(agg, norm_dst, b, w, norm_src, rb):
    _, n_pad, f = agg.shape

    def body(a_ref, nd_ref, b_ref, w_ref, ns_ref, o_ref):
        h = jnp.maximum(
            (a_ref[0] + a_ref[1]) * nd_ref[...] + b_ref[...], 0.0)
        o_ref[...] = jnp.dot(
            h, w_ref[...], precision=lax.Precision.HIGHEST,
            preferred_element_type=jnp.float32) * ns_ref[...]

    return pl.pallas_call(
        body,
        grid=(n_pad // rb,),
        in_specs=[
            pl.BlockSpec((NC, rb, f), lambda i: (0, i, 0)),
            pl.BlockSpec((rb, 1), lambda i: (i, 0)),
            pl.BlockSpec((1, f), lambda i: (0, 0)),
            pl.BlockSpec((f, f), lambda i: (0, 0)),
            pl.BlockSpec((rb, 1), lambda i: (i, 0)),
        ],
        out_specs=pl.BlockSpec((rb, f), lambda i: (i, 0)),
        out_shape=jax.ShapeDtypeStruct((n_pad, f), jnp.float32),
    )(agg, norm_dst, b, w, norm_src)


def _head(agg, norm_dst, b, wd, bd, n_nodes, rb):
    _, n_pad, f = agg.shape
    grid = n_pad // rb

    def body(a_ref, nd_ref, b_ref, wd_ref, bd_ref, o_ref, acc_ref):
        i = pl.program_id(0)
        h = jnp.maximum(
            (a_ref[0] + a_ref[1]) * nd_ref[...] + b_ref[...], 0.0)
        rid = lax.broadcasted_iota(jnp.int32, (rb, f), 0) + i * rb
        h = jnp.where(rid < n_nodes, h, 0.0)
        p = jnp.sum(h, axis=0, keepdims=True)

        @pl.when(i == 0)
        def _():
            acc_ref[...] = p

        @pl.when(i > 0)
        def _():
            acc_ref[...] = acc_ref[...] + p

        @pl.when(i == grid - 1)
        def _():
            g = acc_ref[...] * (1.0 / n_nodes)
            o_ref[...] = jnp.dot(
                g, wd_ref[...], precision=lax.Precision.HIGHEST,
                preferred_element_type=jnp.float32
            ) + bd_ref[...]

    return pl.pallas_call(
        body,
        grid=(grid,),
        in_specs=[
            pl.BlockSpec((NC, rb, f), lambda i: (0, i, 0)),
            pl.BlockSpec((rb, 1), lambda i: (i, 0)),
            pl.BlockSpec((1, f), lambda i: (0, 0)),
            pl.BlockSpec((f, 1), lambda i: (0, 0)),
            pl.BlockSpec((1, 1), lambda i: (0, 0)),
        ],
        out_specs=pl.BlockSpec((1, 1), lambda i: (0, 0)),
        out_shape=jax.ShapeDtypeStruct((1, 1), jnp.float32),
        scratch_shapes=[pltpu.VMEM((1, f), jnp.float32)],
    )(agg, norm_dst, b, wd, bd)


def kernel(in_feat, edge_index, W1, b1, W2, b2, Wd, bd):
    n_nodes, f = in_feat.shape
    n_edges = edge_index.shape[1]

    n_pad = ((n_nodes + 1 + NS * K - 1) // (NS * K)) * (NS * K)
    rb = 1024 if n_pad % 1024 == 0 else NS * K

    def _pad_dst(n):
        return n_nodes + (jnp.arange(n, dtype=jnp.int32)
                          % (n_pad - n_nodes))

    cpt = (n_edges + NW * K - 1) // (NW * K)
    e_pad = NW * K * cpt
    pad_idx = jnp.full((e_pad - n_edges,), n_nodes, jnp.int32)
    src = jnp.concatenate([edge_index[0], pad_idx]).reshape(NW, cpt, K)
    dst = jnp.concatenate(
        [edge_index[1], _pad_dst(e_pad - n_edges)]).reshape(NW, cpt, K)


    deg_part = _make_deg_kernel(cpt, n_pad)(src, dst)
    norms = _norm_kernel(deg_part.reshape(NW, 2, n_pad))
    norm_src = norms[0].reshape(n_pad, 1)
    norm_dst = norms[1].reshape(n_pad, 1)

    mp = _make_mp_kernel(cpt, n_pad, f)

    x_pad = jnp.zeros((n_pad, f), jnp.float32).at[:n_nodes].set(in_feat)
    hs1 = _mm_scale(x_pad, W1, norm_src, n_pad, rb)
    agg1 = mp(hs1, src, dst).reshape(NC, n_pad, f)
    hs2 = _layer2_input(agg1, norm_dst, b1.reshape(1, f), W2, norm_src, rb)
    agg2 = mp(hs2, src, dst).reshape(NC, n_pad, f)
    out = _head(agg2, norm_dst, b2.reshape(1, f), Wd, bd.reshape(1, 1),
                n_nodes, rb)
    return out.reshape(())
